# baseline (speedup 1.0000x reference)
import functools

import jax
import jax.numpy as jnp
from jax import lax
from jax.experimental import pallas as pl
from jax.experimental.pallas import tpu as pltpu
from jax.experimental.pallas import tpu_sc as plsc

_B, _C = 16384, 1000
_K = 12288
_COLS = 1024
_NBLK = _B // _COLS

_NC, _NS = 2, 16
_NW = _NC * _NS
_PER = _B // _NW

_INT_MIN = -2147483648


def _gather_body(flat_ref, idx_ref, out_ref, idx_v, vals_v, sem):
    wid = lax.axis_index("s") * _NC + lax.axis_index("c")
    base = wid * _PER
    pltpu.sync_copy(idx_ref.at[pl.ds(base, _PER)], idx_v)
    pltpu.async_copy(flat_ref.at[idx_v], vals_v, sem).wait()
    pltpu.sync_copy(vals_v, out_ref.at[pl.ds(base, _PER)])


def _sc_gather(flat, idx):
    mesh = plsc.VectorSubcoreMesh(core_axis_name="c", subcore_axis_name="s")
    run = functools.partial(
        pl.kernel,
        mesh=mesh,
        out_type=jax.ShapeDtypeStruct((_B,), jnp.float32),
        scratch_types=[
            pltpu.VMEM((_PER,), jnp.int32),
            pltpu.VMEM((_PER,), jnp.float32),
            pltpu.SemaphoreType.DMA,
        ],
    )(_gather_body)
    return run(flat, idx)


def _logz_body(x_ref, logz_ref):
    x = x_ref[...]
    m = jnp.max(x, axis=0)
    s = jnp.sum(jnp.exp(x - m[None, :]), axis=0)
    logz_ref[...] = (jnp.log(s) + m)[None, :]


def _topk_body(logz_ref, picked_ref, out_ref):
    x = logz_ref[...] - picked_ref[...]
    bits = jax.lax.bitcast_convert_type(x, jnp.int32)
    key = jnp.where(bits >= 0, bits, bits ^ jnp.int32(0x7FFFFFFF))

    def body(i, T):
        cand = T | jax.lax.shift_left(jnp.int32(1), jnp.int32(31) - i)
        scand = cand ^ jnp.int32(_INT_MIN)
        cnt = jnp.sum((key >= scand).astype(jnp.int32))
        return jnp.where(cnt >= _K, cand, T)

    T = jax.lax.fori_loop(0, 32, body, jnp.int32(0))
    kth = T ^ jnp.int32(_INT_MIN)

    gt = key > kth
    cnt_gt = jnp.sum(gt.astype(jnp.int32))
    sum_gt = jnp.sum(jnp.where(gt, x, 0.0))
    tval = jnp.max(jnp.where(key == kth, x, -jnp.inf))
    res = (sum_gt + (_K - cnt_gt).astype(jnp.float32) * tval) / _K
    out_ref[...] = jnp.full((1, 1), res, jnp.float32)


def kernel(input, target):
    xt = input.T
    flat = xt.reshape(-1)
    idx = target.astype(jnp.int32) * jnp.int32(_B) + jnp.arange(_B, dtype=jnp.int32)
    picked = _sc_gather(flat, idx)

    logz = pl.pallas_call(
        _logz_body,
        grid=(_NBLK,),
        in_specs=[pl.BlockSpec((_C, _COLS), lambda i: (0, i))],
        out_specs=pl.BlockSpec((1, _COLS), lambda i: (0, i)),
        out_shape=jax.ShapeDtypeStruct((1, _B), jnp.float32),
    )(xt)

    out = pl.pallas_call(
        _topk_body,
        out_shape=jax.ShapeDtypeStruct((1, 1), jnp.float32),
    )(logz.reshape(128, 128), picked.reshape(128, 128))
    return out[0, 0]


# speedup vs baseline: 1.9772x; 1.9772x over previous
import functools

import jax
import jax.numpy as jnp
from jax import lax
from jax.experimental import pallas as pl
from jax.experimental.pallas import tpu as pltpu
from jax.experimental.pallas import tpu_sc as plsc

_B, _C = 16384, 1000
_K = 12288
_CHI, _IHI, _CLO, _ILO = 125, 128, 8, 128
_BI = 8
_NBLK = _IHI // _BI

_NC, _NS = 2, 16
_NW = _NC * _NS
_PER = _B // _NW

_INT_MIN = -2147483648


def _gather_body(flat_ref, idx_ref, out_ref, idx_v, vals_v, sem):
    wid = lax.axis_index("s") * _NC + lax.axis_index("c")
    base = wid * _PER
    pltpu.sync_copy(idx_ref.at[pl.ds(base, _PER)], idx_v)
    pltpu.async_copy(flat_ref.at[idx_v], vals_v, sem).wait()
    pltpu.sync_copy(vals_v, out_ref.at[pl.ds(base, _PER)])


def _sc_gather(flat, idx):
    mesh = plsc.VectorSubcoreMesh(core_axis_name="c", subcore_axis_name="s")
    run = functools.partial(
        pl.kernel,
        mesh=mesh,
        out_type=jax.ShapeDtypeStruct((_B,), jnp.float32),
        scratch_types=[
            pltpu.VMEM((_PER,), jnp.int32),
            pltpu.VMEM((_PER,), jnp.float32),
            pltpu.SemaphoreType.DMA,
        ],
    )(_gather_body)
    return run(flat, idx)


def _logz_body(x_ref, logz_ref):
    x = x_ref[...]
    m1 = jnp.max(x, axis=0)
    m = jnp.max(m1, axis=1)
    e = jnp.exp(x - m[None, :, None, :])
    s = jnp.sum(jnp.sum(e, axis=0), axis=1)
    logz_ref[...] = jnp.log(s) + m


def _topk_body(logz_ref, picked_ref, out_ref):
    x = logz_ref[...] - picked_ref[...]
    bits = jax.lax.bitcast_convert_type(x, jnp.int32)
    key = jnp.where(bits >= 0, bits, bits ^ jnp.int32(0x7FFFFFFF))

    def body(i, T):
        cand = T | jax.lax.shift_left(jnp.int32(1), jnp.int32(31) - i)
        scand = cand ^ jnp.int32(_INT_MIN)
        cnt = jnp.sum((key >= scand).astype(jnp.int32))
        return jnp.where(cnt >= _K, cand, T)

    T = jax.lax.fori_loop(0, 32, body, jnp.int32(0))
    kth = T ^ jnp.int32(_INT_MIN)

    gt = key > kth
    cnt_gt = jnp.sum(gt.astype(jnp.int32))
    sum_gt = jnp.sum(jnp.where(gt, x, 0.0))
    tval = jnp.max(jnp.where(key == kth, x, -jnp.inf))
    res = (sum_gt + (_K - cnt_gt).astype(jnp.float32) * tval) / _K
    out_ref[...] = jnp.full((1, 1), res, jnp.float32)


def kernel(input, target):
    x4 = input.T.reshape(_CHI, _CLO, _IHI, _ILO).transpose(0, 2, 1, 3)
    flat = x4.reshape(-1)

    t = target.astype(jnp.int32)
    iar = jnp.arange(_B, dtype=jnp.int32)
    idx = ((t // _CLO) * _IHI + iar // _ILO) * (_CLO * _ILO) \
        + (t % _CLO) * _ILO + (iar % _ILO)
    picked = _sc_gather(flat, idx)

    logz = pl.pallas_call(
        _logz_body,
        grid=(_NBLK,),
        in_specs=[pl.BlockSpec((_CHI, _BI, _CLO, _ILO), lambda b: (0, b, 0, 0))],
        out_specs=pl.BlockSpec((_BI, _ILO), lambda b: (b, 0)),
        out_shape=jax.ShapeDtypeStruct((_IHI, _ILO), jnp.float32),
    )(x4)

    out = pl.pallas_call(
        _topk_body,
        out_shape=jax.ShapeDtypeStruct((1, 1), jnp.float32),
    )(logz, picked.reshape(_IHI, _ILO))
    return out[0, 0]


# speedup vs baseline: 2.1090x; 1.0666x over previous
import functools

import jax
import jax.numpy as jnp
from jax import lax
from jax.experimental import pallas as pl
from jax.experimental.pallas import tpu as pltpu
from jax.experimental.pallas import tpu_sc as plsc

_B, _C = 16384, 1000
_K = 12288
_CHI, _IHI, _CLO, _ILO = 125, 128, 8, 128
_BI = 16
_NBLK = _IHI // _BI

_NC, _NS = 2, 16
_NW = _NC * _NS
_PER = _B // _NW

_INT_MIN = -2147483648


def _gather_body(flat_ref, idx_ref, out_ref, idx_v, vals_v, sem):
    wid = lax.axis_index("s") * _NC + lax.axis_index("c")
    base = wid * _PER
    pltpu.sync_copy(idx_ref.at[pl.ds(base, _PER)], idx_v)
    pltpu.async_copy(flat_ref.at[idx_v], vals_v, sem).wait()
    pltpu.sync_copy(vals_v, out_ref.at[pl.ds(base, _PER)])


def _sc_gather(flat, idx):
    mesh = plsc.VectorSubcoreMesh(core_axis_name="c", subcore_axis_name="s")
    run = functools.partial(
        pl.kernel,
        mesh=mesh,
        out_type=jax.ShapeDtypeStruct((_B,), jnp.float32),
        scratch_types=[
            pltpu.VMEM((_PER,), jnp.int32),
            pltpu.VMEM((_PER,), jnp.float32),
            pltpu.SemaphoreType.DMA,
        ],
    )(_gather_body)
    return run(flat, idx)


def _logz_body(x_ref, logz_ref):
    x = x_ref[...]
    m1 = jnp.max(x, axis=0)
    m = jnp.max(m1, axis=1)
    e = jnp.exp(x - m[None, :, None, :])
    s = jnp.sum(jnp.sum(e, axis=0), axis=1)
    logz_ref[...] = jnp.log(s) + m


def _topk_body(logz_ref, picked_ref, out_ref):
    x = logz_ref[...] - picked_ref[...]
    bits = jax.lax.bitcast_convert_type(x, jnp.int32)
    key = jnp.where(bits >= 0, bits, bits ^ jnp.int32(0x7FFFFFFF))

    def body(i, T):
        cand = T | jax.lax.shift_left(jnp.int32(1), jnp.int32(31) - i)
        scand = cand ^ jnp.int32(_INT_MIN)
        cnt = jnp.sum((key >= scand).astype(jnp.int32))
        return jnp.where(cnt >= _K, cand, T)

    T = jax.lax.fori_loop(0, 32, body, jnp.int32(0))
    kth = T ^ jnp.int32(_INT_MIN)

    gt = key > kth
    cnt_gt = jnp.sum(gt.astype(jnp.int32))
    sum_gt = jnp.sum(jnp.where(gt, x, 0.0))
    tval = jnp.max(jnp.where(key == kth, x, -jnp.inf))
    res = (sum_gt + (_K - cnt_gt).astype(jnp.float32) * tval) / _K
    out_ref[...] = jnp.full((1, 1), res, jnp.float32)


def kernel(input, target):
    x4 = input.T.reshape(_CHI, _CLO, _IHI, _ILO).transpose(0, 2, 1, 3)
    flat = x4.reshape(-1)

    t = target.astype(jnp.int32)
    iar = jnp.arange(_B, dtype=jnp.int32)
    idx = ((t // _CLO) * _IHI + iar // _ILO) * (_CLO * _ILO) \
        + (t % _CLO) * _ILO + (iar % _ILO)
    picked = _sc_gather(flat, idx)

    logz = pl.pallas_call(
        _logz_body,
        grid=(_NBLK,),
        in_specs=[pl.BlockSpec((_CHI, _BI, _CLO, _ILO), lambda b: (0, b, 0, 0))],
        out_specs=pl.BlockSpec((_BI, _ILO), lambda b: (b, 0)),
        out_shape=jax.ShapeDtypeStruct((_IHI, _ILO), jnp.float32),
    )(x4)

    out = pl.pallas_call(
        _topk_body,
        out_shape=jax.ShapeDtypeStruct((1, 1), jnp.float32),
    )(logz, picked.reshape(_IHI, _ILO))
    return out[0, 0]
